# baseline (speedup 1.0000x reference)
import functools

import jax
import jax.numpy as jnp
from jax.experimental import pallas as pl
from jax.experimental.pallas import tpu as pltpu

_BF = jnp.bfloat16
_F32 = jnp.float32


def _dot(a, b):
    return jnp.dot(a, b, preferred_element_type=_F32)


def _w(wpack, i):
    return wpack[pl.ds(128 * i, 128), :]


def _b(bpack, i):
    return bpack[pl.ds(i, 1), :]


def _compute_projs(proj, hd, hp, wpack, bpack, w0):
    hdb = hd.astype(_BF)
    hpb = hp.astype(_BF)
    proj[0] = (_dot(hdb, _w(wpack, w0).astype(_BF)) + _b(bpack, w0)).astype(_BF)
    proj[1] = (_dot(hpb, _w(wpack, w0 + 1).astype(_BF)) + _b(bpack, w0 + 1)).astype(_BF)
    proj[2] = (_dot(hdb, _w(wpack, w0 + 2).astype(_BF)) + _b(bpack, w0 + 2)).astype(_BF)
    proj[3] = (_dot(hpb, _w(wpack, w0 + 3).astype(_BF)) + _b(bpack, w0 + 3)).astype(_BF)


def _tiles(a_dd, a_dp, a_pd, a_pp, proj, hd_t, hp_t, wpack, bpack, ws0):
    acc_d = (_dot(a_dd[0].astype(_BF), proj[0])
             + _dot(a_dp[0].astype(_BF), proj[1])
             + _dot(hd_t, _w(wpack, ws0)) + _b(bpack, ws0))
    acc_p = (_dot(a_pd[0].astype(_BF), proj[2])
             + _dot(a_pp[0].astype(_BF), proj[3])
             + _dot(hp_t, _w(wpack, ws0 + 1)) + _b(bpack, ws0 + 1))
    return jnp.maximum(acc_d, 0.0), jnp.maximum(acc_p, 0.0)


def _fused_kernel(base_d, base_p, fx,
                  a0dd, a0dp, a0pd, a0pp,
                  a1dd, a1dp, a1pd, a1pp,
                  wpack, bpack,
                  od, op_, h1d, h1p, h0d, proj, *, R, TT):
    t = pl.program_id(0)

    @pl.when(t == 0)
    def _init_l1():
        h0 = base_d[...] + _dot(fx[...], _w(wpack, 0)) + _b(bpack, 0)
        h0d[...] = h0
        _compute_projs(proj, h0, base_p[...], wpack, bpack, 1)

    @pl.when(t < TT)
    def _layer1_tile():
        row0 = t * R
        hd_t, hp_t = (h0d[pl.ds(row0, R), :], base_p[pl.ds(row0, R), :])
        out_d, out_p = _tiles(a0dd, a0dp, a0pd, a0pp, proj, hd_t, hp_t,
                              wpack, bpack, 5)
        h1d[pl.ds(row0, R), :] = out_d
        h1p[pl.ds(row0, R), :] = out_p

    @pl.when(t == TT)
    def _init_l2():
        _compute_projs(proj, h1d[...], h1p[...], wpack, bpack, 7)

    @pl.when(t >= TT)
    def _layer2_tile():
        row0 = (t - TT) * R
        out_d, out_p = _tiles(a1dd, a1dp, a1pd, a1pp, proj,
                              h1d[pl.ds(row0, R), :], h1p[pl.ds(row0, R), :],
                              wpack, bpack, 11)
        od[...] = out_d[:, :16]
        op_[...] = out_p[:, :16]


def kernel(base_drug, base_protein,
           feat_drug_x, feat_drug_w, feat_drug_b,
           conv0_drug_a, conv0_drug_w, conv0_drug_b, conv0_drug_wself, conv0_drug_bself,
           conv0_protein_a, conv0_protein_w, conv0_protein_b, conv0_protein_wself, conv0_protein_bself,
           conv1_drug_a, conv1_drug_w, conv1_drug_b, conv1_drug_wself, conv1_drug_bself,
           conv1_protein_a, conv1_protein_w, conv1_protein_b, conv1_protein_wself, conv1_protein_bself):
    n = conv0_drug_a.shape[1]
    d = conv0_drug_w.shape[2]
    r = 256 if n % 256 == 0 else n
    tt = n // r

    wpack = jnp.concatenate([
        feat_drug_w,
        conv0_drug_w[0], conv0_drug_w[1],
        conv0_protein_w[0], conv0_protein_w[1],
        conv0_drug_wself, conv0_protein_wself,
        conv1_drug_w[0], conv1_drug_w[1],
        conv1_protein_w[0], conv1_protein_w[1],
        conv1_drug_wself, conv1_protein_wself], axis=0)
    bpack = jnp.concatenate([
        feat_drug_b,
        conv0_drug_b[0], conv0_drug_b[1],
        conv0_protein_b[0], conv0_protein_b[1],
        conv0_drug_bself, conv0_protein_bself,
        conv1_drug_b[0], conv1_drug_b[1],
        conv1_protein_b[0], conv1_protein_b[1],
        conv1_drug_bself, conv1_protein_bself,
        jnp.zeros((3, d), _F32)], axis=0)

    whole = lambda shape: pl.BlockSpec(shape, lambda t: (0,) * len(shape))

    def a0_spec(e, cb):
        return pl.BlockSpec(
            (1, r, n),
            lambda t, e=e, cb=cb: (e, jnp.minimum(t, tt - 1), cb))

    def a1_spec(e, cb):
        return pl.BlockSpec(
            (1, r, n),
            lambda t, e=e, cb=cb: (e, jnp.maximum(t - tt, 0), cb))

    out_spec = pl.BlockSpec((r, 16), lambda t: (jnp.maximum(t - tt, 0), 0))

    flops = 2 * 8 * n * n * d + 4 * (8 * n * d * d + 2 * n * d * d)
    bytes_ = 4 * (8 * n * n + 5 * n * d + 16 * d * d)
    ins = [base_drug, base_protein, feat_drug_x,
           conv0_drug_a, conv0_drug_a, conv0_protein_a, conv0_protein_a,
           conv1_drug_a, conv1_drug_a, conv1_protein_a, conv1_protein_a,
           wpack, bpack]
    in_specs = ([whole(x.shape) for x in ins[:3]]
                + [a0_spec(0, 0), a0_spec(1, 1), a0_spec(0, 0), a0_spec(1, 1)]
                + [a1_spec(0, 0), a1_spec(1, 1), a1_spec(0, 0), a1_spec(1, 1)]
                + [whole(wpack.shape), whole(bpack.shape)])
    h2d, h2p = pl.pallas_call(
        functools.partial(_fused_kernel, R=r, TT=tt),
        grid=(2 * tt,),
        in_specs=in_specs,
        out_specs=[out_spec, out_spec],
        out_shape=[jax.ShapeDtypeStruct((n, 16), _F32)] * 2,
        scratch_shapes=[pltpu.VMEM((n, d), _F32), pltpu.VMEM((n, d), _F32),
                        pltpu.VMEM((n, d), _F32), pltpu.VMEM((4, n, d), _BF)],
        compiler_params=pltpu.CompilerParams(
            dimension_semantics=("arbitrary",)),
        cost_estimate=pl.CostEstimate(flops=flops, transcendentals=0,
                                      bytes_accessed=bytes_),
    )(*ins)
    return {"drug": h2d, "protein": h2p}


# speedup vs baseline: 1.4391x; 1.4391x over previous
import functools

import jax
import jax.numpy as jnp
from jax.experimental import pallas as pl
from jax.experimental.pallas import tpu as pltpu

_BF = jnp.bfloat16
_F32 = jnp.float32


def _dot(a, b):
    return jnp.dot(a, b, preferred_element_type=_F32)


def _compute_projs(proj, hd, hp, wd, bd, wp, bp):
    hdb = hd.astype(_BF)
    hpb = hp.astype(_BF)
    proj[0] = (_dot(hdb, wd[0].astype(_BF)) + bd[0]).astype(_BF)
    proj[1] = (_dot(hpb, wd[1].astype(_BF)) + bd[1]).astype(_BF)
    proj[2] = (_dot(hdb, wp[0].astype(_BF)) + bp[0]).astype(_BF)
    proj[3] = (_dot(hpb, wp[1].astype(_BF)) + bp[1]).astype(_BF)


def _tiles(a_dd, a_dp, a_pd, a_pp, proj, hd_t, hp_t, wsd, bsd, wsp, bsp):
    acc_d = (_dot(a_dd[0].astype(_BF), proj[0])
             + _dot(a_dp[0].astype(_BF), proj[1])
             + _dot(hd_t, wsd[...]) + bsd[...])
    acc_p = (_dot(a_pd[0].astype(_BF), proj[2])
             + _dot(a_pp[0].astype(_BF), proj[3])
             + _dot(hp_t, wsp[...]) + bsp[...])
    return jnp.maximum(acc_d, 0.0), jnp.maximum(acc_p, 0.0)


def _fused_kernel(base_d, base_p, fx, fw, fb,
                  a0dd, a0dp, a0pd, a0pp,
                  w0d, b0d, w0p, b0p, ws0d, bs0d, ws0p, bs0p,
                  a1dd, a1dp, a1pd, a1pp,
                  w1d, b1d, w1p, b1p, ws1d, bs1d, ws1p, bs1p,
                  od, op_, h1d, h1p, h0d, proj, *, R, TT):
    t = pl.program_id(0)

    @pl.when(t == 0)
    def _init_l1():
        h0 = base_d[...] + _dot(fx[...], fw[...]) + fb[...]
        h0d[...] = h0
        _compute_projs(proj, h0, base_p[...], w0d, b0d, w0p, b0p)

    @pl.when(t < TT)
    def _layer1_tile():
        row0 = t * R
        hd_t, hp_t = (h0d[pl.ds(row0, R), :], base_p[pl.ds(row0, R), :])
        out_d, out_p = _tiles(a0dd, a0dp, a0pd, a0pp, proj, hd_t, hp_t,
                              ws0d, bs0d, ws0p, bs0p)
        h1d[pl.ds(row0, R), :] = out_d
        h1p[pl.ds(row0, R), :] = out_p

    @pl.when(t == TT)
    def _init_l2():
        _compute_projs(proj, h1d[...], h1p[...], w1d, b1d, w1p, b1p)

    @pl.when(t >= TT)
    def _layer2_tile():
        row0 = (t - TT) * R
        out_d, out_p = _tiles(a1dd, a1dp, a1pd, a1pp, proj,
                              h1d[pl.ds(row0, R), :], h1p[pl.ds(row0, R), :],
                              ws1d, bs1d, ws1p, bs1p)
        od[...] = out_d[:, :16]
        op_[...] = out_p[:, :16]


def kernel(base_drug, base_protein,
           feat_drug_x, feat_drug_w, feat_drug_b,
           conv0_drug_a, conv0_drug_w, conv0_drug_b, conv0_drug_wself, conv0_drug_bself,
           conv0_protein_a, conv0_protein_w, conv0_protein_b, conv0_protein_wself, conv0_protein_bself,
           conv1_drug_a, conv1_drug_w, conv1_drug_b, conv1_drug_wself, conv1_drug_bself,
           conv1_protein_a, conv1_protein_w, conv1_protein_b, conv1_protein_wself, conv1_protein_bself):
    n = conv0_drug_a.shape[1]
    d = conv0_drug_w.shape[2]
    r = 256 if n % 256 == 0 else n
    tt = n // r

    whole = lambda shape: pl.BlockSpec(shape, lambda t: (0,) * len(shape))

    def a0_spec(e, cb):
        return pl.BlockSpec(
            (1, r, n),
            lambda t, e=e, cb=cb: (e, jnp.minimum(t, tt - 1), cb))

    def a1_spec(e, cb):
        return pl.BlockSpec(
            (1, r, n),
            lambda t, e=e, cb=cb: (e, jnp.maximum(t - tt, 0), cb))

    out_spec = pl.BlockSpec((r, 16), lambda t: (jnp.maximum(t - tt, 0), 0))

    flops = 2 * 8 * n * n * d + 4 * (8 * n * d * d + 2 * n * d * d)
    bytes_ = 4 * (8 * n * n + 5 * n * d + 16 * d * d)
    ins = [base_drug, base_protein, feat_drug_x, feat_drug_w, feat_drug_b,
           conv0_drug_a, conv0_drug_a, conv0_protein_a, conv0_protein_a,
           conv0_drug_w, conv0_drug_b, conv0_protein_w, conv0_protein_b,
           conv0_drug_wself, conv0_drug_bself, conv0_protein_wself, conv0_protein_bself,
           conv1_drug_a, conv1_drug_a, conv1_protein_a, conv1_protein_a,
           conv1_drug_w, conv1_drug_b, conv1_protein_w, conv1_protein_b,
           conv1_drug_wself, conv1_drug_bself, conv1_protein_wself, conv1_protein_bself]
    in_specs = ([whole(x.shape) for x in ins[:5]]
                + [a0_spec(0, 0), a0_spec(1, 1), a0_spec(0, 0), a0_spec(1, 1)]
                + [whole(x.shape) for x in ins[9:17]]
                + [a1_spec(0, 0), a1_spec(1, 1), a1_spec(0, 0), a1_spec(1, 1)]
                + [whole(x.shape) for x in ins[21:]])
    h2d, h2p = pl.pallas_call(
        functools.partial(_fused_kernel, R=r, TT=tt),
        grid=(2 * tt,),
        in_specs=in_specs,
        out_specs=[out_spec, out_spec],
        out_shape=[jax.ShapeDtypeStruct((n, 16), _F32)] * 2,
        scratch_shapes=[pltpu.VMEM((n, d), _F32), pltpu.VMEM((n, d), _F32),
                        pltpu.VMEM((n, d), _F32), pltpu.VMEM((4, n, d), _BF)],
        compiler_params=pltpu.CompilerParams(
            dimension_semantics=("arbitrary",)),
        cost_estimate=pl.CostEstimate(flops=flops, transcendentals=0,
                                      bytes_accessed=bytes_),
    )(*ins)
    return {"drug": h2d, "protein": h2p}
